# Initial kernel scaffold; baseline (speedup 1.0000x reference)
import jax
import jax.numpy as jnp
from jax.experimental import pallas as pl


def kernel(x, edge_index, edge_attr, W1, b1, W2, b2, Wfc, bfc):
    raise NotImplementedError("write your pallas kernel here")


# speedup vs baseline: 8.6090x; 8.6090x over previous
import functools

import jax
import jax.numpy as jnp
from jax import lax
from jax.experimental import pallas as pl
from jax.experimental.pallas import tpu as pltpu
from jax.experimental.pallas import tpu_sc as plsc

N_NODES = 10000
N_PAD = 10240
N_EDGES = 320000
HID = 64
DEGW = 16
NC = 2
NS = 16
EDGES_PER_TILE = N_EDGES // (NC * NS)
CHUNK = 80
NCHUNKS = EDGES_PER_TILE // CHUNK
ROWS_PER_TILE = N_PAD // NS

_mesh = plsc.VectorSubcoreMesh(core_axis_name="c", subcore_axis_name="s")


@functools.partial(
    pl.kernel,
    out_type=jax.ShapeDtypeStruct((NC, N_PAD, 128), jnp.float32),
    mesh=_mesh,
    scratch_types=[
        pltpu.VMEM((CHUNK,), jnp.int32),
        pltpu.VMEM((CHUNK + 16,), jnp.float32),
        pltpu.VMEM((CHUNK, 128), jnp.float32),
        pltpu.VMEM_SHARED((N_PAD, 128), jnp.float32),
    ],
)
def _deg_kernel(dst_hbm, ew_hbm, out_hbm, dst_v, ew_v, val_v, acc):
    c = lax.axis_index("c")
    s = lax.axis_index("s")
    wid = s * NC + c
    base = s * ROWS_PER_TILE
    zeros_f = jnp.zeros((16,), jnp.float32)
    lane0 = lax.iota(jnp.int32, 16) == 0

    def zero_val(e, carry):
        for j in range(8):
            val_v[e, pl.ds(j * 16, 16)] = zeros_f
        return carry

    lax.fori_loop(0, CHUNK, zero_val, 0)
    for k in range(ROWS_PER_TILE // CHUNK):
        pltpu.sync_copy(val_v, acc.at[pl.ds(base + k * CHUNK, CHUNK)])
    plsc.subcore_barrier()

    def body(i, carry):
        off = wid * EDGES_PER_TILE + i * CHUNK
        pltpu.sync_copy(dst_hbm.at[pl.ds(off, CHUNK)], dst_v)
        pltpu.sync_copy(ew_hbm.at[pl.ds(off, CHUNK)], ew_v.at[pl.ds(0, CHUNK)])

        def fill_body(e, carry2):
            w = ew_v[pl.ds(e, 16)][0]
            val_v[e, pl.ds(0, 16)] = jnp.where(lane0, zeros_f + w, zeros_f)
            return carry2

        lax.fori_loop(0, CHUNK, fill_body, 0)
        pltpu.sync_copy(val_v, acc.at[dst_v], add=True)
        return carry

    lax.fori_loop(0, NCHUNKS, body, 0)
    plsc.subcore_barrier()
    pltpu.sync_copy(acc.at[pl.ds(base, ROWS_PER_TILE)],
                    out_hbm.at[c].at[pl.ds(base, ROWS_PER_TILE)])


@functools.partial(
    pl.kernel,
    out_type=jax.ShapeDtypeStruct((NC, N_PAD, 128), jnp.float32),
    mesh=_mesh,
    scratch_types=[
        pltpu.VMEM((CHUNK,), jnp.int32),
        pltpu.VMEM((CHUNK,), jnp.int32),
        pltpu.VMEM((CHUNK + 16,), jnp.float32),
        pltpu.VMEM((CHUNK, 128), jnp.float32),
        pltpu.VMEM((CHUNK, 128), jnp.float32),
        pltpu.VMEM_SHARED((N_PAD, 128), jnp.float32),
        pltpu.SemaphoreType.DMA,
    ],
)
def _agg_kernel(y_hbm, src_hbm, dst_hbm, ew_hbm, out_hbm,
                src_v, dst_v, ew_v, rows_v, val_v, acc, sem):
    c = lax.axis_index("c")
    s = lax.axis_index("s")
    wid = s * NC + c
    base = s * ROWS_PER_TILE
    zeros_f = jnp.zeros((16,), jnp.float32)

    def zero_val(e, carry):
        for j in range(8):
            val_v[e, pl.ds(j * 16, 16)] = zeros_f
        return carry

    lax.fori_loop(0, CHUNK, zero_val, 0)
    for k in range(ROWS_PER_TILE // CHUNK):
        pltpu.sync_copy(val_v, acc.at[pl.ds(base + k * CHUNK, CHUNK)])
    plsc.subcore_barrier()

    def chunk_body(i, carry):
        off = wid * EDGES_PER_TILE + i * CHUNK
        pltpu.sync_copy(src_hbm.at[pl.ds(off, CHUNK)], src_v)
        pltpu.sync_copy(dst_hbm.at[pl.ds(off, CHUNK)], dst_v)
        pltpu.sync_copy(ew_hbm.at[pl.ds(off, CHUNK)], ew_v.at[pl.ds(0, CHUNK)])
        pltpu.async_copy(y_hbm.at[src_v], rows_v, sem).wait()

        def scale_body(e, carry2):
            w = zeros_f + ew_v[pl.ds(e, 16)][0]
            for j in range(HID // 16):
                val_v[e, pl.ds(j * 16, 16)] = rows_v[e, pl.ds(j * 16, 16)] * w
            return carry2

        lax.fori_loop(0, CHUNK, scale_body, 0)
        pltpu.sync_copy(val_v, acc.at[dst_v], add=True)
        return carry

    lax.fori_loop(0, NCHUNKS, chunk_body, 0)
    plsc.subcore_barrier()
    pltpu.sync_copy(acc.at[pl.ds(base, ROWS_PER_TILE)],
                    out_hbm.at[c].at[pl.ds(base, ROWS_PER_TILE)])


def _dinv_from(degp_ref):
    deg = degp_ref[0][:, 0:1] + degp_ref[1][:, 0:1] + 1.0
    return lax.rsqrt(deg)


def _tc_a_body(degp_ref, x_ref, w1_ref, y1_ref):
    dinv = _dinv_from(degp_ref)
    xw = jnp.dot(x_ref[...], w1_ref[...], preferred_element_type=jnp.float32)
    y = dinv * xw
    y1_ref[...] = jnp.concatenate([y, jnp.zeros_like(y)], axis=1)


def _tc_b_body(degp_ref, p_ref, y_ref, b_ref, w_ref, out_ref):
    dinv = _dinv_from(degp_ref)
    y = y_ref[...][:, 0:HID]
    agg = p_ref[0][:, 0:HID] + p_ref[1][:, 0:HID]
    h = jnp.maximum(dinv * (agg + y) + b_ref[...], 0.0)
    y2 = dinv * jnp.dot(h, w_ref[...], preferred_element_type=jnp.float32)
    out_ref[...] = jnp.concatenate([y2, jnp.zeros_like(y2)], axis=1)


def _tc_c_body(degp_ref, q_ref, y_ref, b_ref, wfc_ref, bfc_ref, out_ref):
    dinv = _dinv_from(degp_ref)
    y = y_ref[...][:, 0:HID]
    agg = q_ref[0][:, 0:HID] + q_ref[1][:, 0:HID]
    h = jnp.maximum(dinv * (agg + y) + b_ref[...], 0.0)
    out_ref[...] = jnp.dot(h, wfc_ref[...],
                           preferred_element_type=jnp.float32) + bfc_ref[...]


_BN = 2000
_GRID = N_NODES // _BN


def _row_spec(width):
    return pl.BlockSpec((_BN, width), lambda i: (i, 0))


def _pair_spec(width):
    return pl.BlockSpec((NC, _BN, width), lambda i: (0, i, 0))


def _full_spec(shape):
    return pl.BlockSpec(shape, lambda i: tuple(0 for _ in shape))


def kernel(x, edge_index, edge_attr, W1, b1, W2, b2, Wfc, bfc):
    src = edge_index[0]
    dst = edge_index[1]

    degp = _deg_kernel(dst, edge_attr)

    y1 = pl.pallas_call(
        _tc_a_body,
        grid=(_GRID,),
        in_specs=[_pair_spec(128), _row_spec(128), _full_spec(W1.shape)],
        out_specs=_row_spec(2 * HID),
        out_shape=jax.ShapeDtypeStruct((N_NODES, 2 * HID), jnp.float32),
    )(degp, x, W1)

    p = _agg_kernel(y1, src, dst, edge_attr)

    y2 = pl.pallas_call(
        _tc_b_body,
        grid=(_GRID,),
        in_specs=[_pair_spec(128), _pair_spec(128), _row_spec(2 * HID),
                  _full_spec((1, HID)), _full_spec(W2.shape)],
        out_specs=_row_spec(2 * HID),
        out_shape=jax.ShapeDtypeStruct((N_NODES, 2 * HID), jnp.float32),
    )(degp, p, y1, b1.reshape(1, HID), W2)

    q = _agg_kernel(y2, src, dst, edge_attr)

    out = pl.pallas_call(
        _tc_c_body,
        grid=(_GRID,),
        in_specs=[_pair_spec(128), _pair_spec(128), _row_spec(2 * HID),
                  _full_spec((1, HID)), _full_spec(Wfc.shape),
                  _full_spec((1, Wfc.shape[1]))],
        out_specs=_row_spec(Wfc.shape[1]),
        out_shape=jax.ShapeDtypeStruct((N_NODES, Wfc.shape[1]), jnp.float32),
    )(degp, q, y2, b2.reshape(1, HID), Wfc, bfc.reshape(1, -1))

    return out


# speedup vs baseline: 11.0334x; 1.2816x over previous
import functools

import jax
import jax.numpy as jnp
from jax import lax
from jax.experimental import pallas as pl
from jax.experimental.pallas import tpu as pltpu
from jax.experimental.pallas import tpu_sc as plsc

N_NODES = 10000
N_PAD = 10240
N_EDGES = 320000
HID = 64
DEGW = 16
NC = 2
NS = 16
EDGES_PER_TILE = N_EDGES // (NC * NS)
CHUNK = 80
NCHUNKS = EDGES_PER_TILE // CHUNK
ROWS_PER_TILE = N_PAD // NS

_mesh = plsc.VectorSubcoreMesh(core_axis_name="c", subcore_axis_name="s")


@functools.partial(
    pl.kernel,
    out_type=jax.ShapeDtypeStruct((NC, N_PAD, 128), jnp.float32),
    mesh=_mesh,
    scratch_types=[
        pltpu.VMEM((CHUNK,), jnp.int32),
        pltpu.VMEM((CHUNK + 16,), jnp.float32),
        pltpu.VMEM((CHUNK, 128), jnp.float32),
        pltpu.VMEM_SHARED((N_PAD, 128), jnp.float32),
    ],
)
def _deg_kernel(dst_hbm, ew_hbm, out_hbm, dst_v, ew_v, val_v, acc):
    c = lax.axis_index("c")
    s = lax.axis_index("s")
    wid = s * NC + c
    base = s * ROWS_PER_TILE
    zeros_f = jnp.zeros((16,), jnp.float32)
    lane0 = lax.iota(jnp.int32, 16) == 0

    def zero_val(e, carry):
        for j in range(8):
            val_v[e, pl.ds(j * 16, 16)] = zeros_f
        return carry

    lax.fori_loop(0, CHUNK, zero_val, 0)
    for k in range(ROWS_PER_TILE // CHUNK):
        pltpu.sync_copy(val_v, acc.at[pl.ds(base + k * CHUNK, CHUNK)])
    plsc.subcore_barrier()

    def body(i, carry):
        off = wid * EDGES_PER_TILE + i * CHUNK
        pltpu.sync_copy(dst_hbm.at[pl.ds(off, CHUNK)], dst_v)
        pltpu.sync_copy(ew_hbm.at[pl.ds(off, CHUNK)], ew_v.at[pl.ds(0, CHUNK)])

        def fill_body(e, carry2):
            w = ew_v[pl.ds(e, 16)][0]
            val_v[e, pl.ds(0, 16)] = jnp.where(lane0, zeros_f + w, zeros_f)
            return carry2

        lax.fori_loop(0, CHUNK, fill_body, 0)
        pltpu.sync_copy(val_v, acc.at[dst_v], add=True)
        return carry

    lax.fori_loop(0, NCHUNKS, body, 0)
    plsc.subcore_barrier()
    pltpu.sync_copy(acc.at[pl.ds(base, ROWS_PER_TILE)],
                    out_hbm.at[c].at[pl.ds(base, ROWS_PER_TILE)])


@functools.partial(
    pl.kernel,
    out_type=jax.ShapeDtypeStruct((NC, N_PAD, 128), jnp.float32),
    mesh=_mesh,
    scratch_types=[
        pltpu.VMEM((CHUNK,), jnp.int32),
        pltpu.VMEM((CHUNK,), jnp.int32),
        pltpu.VMEM((CHUNK,), jnp.int32),
        pltpu.VMEM((CHUNK,), jnp.int32),
        pltpu.VMEM((CHUNK + 16,), jnp.float32),
        pltpu.VMEM((CHUNK + 16,), jnp.float32),
        pltpu.VMEM((CHUNK, 128), jnp.float32),
        pltpu.VMEM((CHUNK, 128), jnp.float32),
        pltpu.VMEM((CHUNK, 128), jnp.float32),
        pltpu.VMEM_SHARED((N_PAD, 128), jnp.float32),
        pltpu.SemaphoreType.DMA,
        pltpu.SemaphoreType.DMA,
    ],
)
def _agg_kernel(y_hbm, src_hbm, dst_hbm, ew_hbm, out_hbm,
                src_a, src_b, dst_a, dst_b, ew_a, ew_b,
                rows_a, rows_b, val_v, acc, sem_a, sem_b):
    c = lax.axis_index("c")
    s = lax.axis_index("s")
    wid = s * NC + c
    base = s * ROWS_PER_TILE
    zeros_f = jnp.zeros((16,), jnp.float32)

    def zero_val(e, carry):
        for j in range(8):
            val_v[e, pl.ds(j * 16, 16)] = zeros_f
        return carry

    lax.fori_loop(0, CHUNK, zero_val, 0)
    for k in range(ROWS_PER_TILE // CHUNK):
        pltpu.sync_copy(val_v, acc.at[pl.ds(base + k * CHUNK, CHUNK)])
    plsc.subcore_barrier()

    def prefetch(i, src_v, dst_v, ew_v, rows_v, sem):
        off = wid * EDGES_PER_TILE + i * CHUNK
        pltpu.sync_copy(src_hbm.at[pl.ds(off, CHUNK)], src_v)
        pltpu.sync_copy(dst_hbm.at[pl.ds(off, CHUNK)], dst_v)
        pltpu.sync_copy(ew_hbm.at[pl.ds(off, CHUNK)], ew_v.at[pl.ds(0, CHUNK)])
        pltpu.make_async_copy(y_hbm.at[src_v], rows_v, sem).start()

    def consume(src_v, dst_v, ew_v, rows_v, sem):
        pltpu.make_async_copy(y_hbm.at[src_v], rows_v, sem).wait()

        def scale_body(e, carry2):
            w = zeros_f + ew_v[pl.ds(e, 16)][0]
            for j in range(HID // 16):
                val_v[e, pl.ds(j * 16, 16)] = rows_v[e, pl.ds(j * 16, 16)] * w
            return carry2

        lax.fori_loop(0, CHUNK, scale_body, 0)
        pltpu.sync_copy(val_v, acc.at[dst_v], add=True)

    prefetch(0, src_a, dst_a, ew_a, rows_a, sem_a)

    def pair_body(j, carry):
        prefetch(2 * j + 1, src_b, dst_b, ew_b, rows_b, sem_b)
        consume(src_a, dst_a, ew_a, rows_a, sem_a)
        prefetch(2 * j + 2, src_a, dst_a, ew_a, rows_a, sem_a)
        consume(src_b, dst_b, ew_b, rows_b, sem_b)
        return carry

    lax.fori_loop(0, NCHUNKS // 2, pair_body, 0)
    consume(src_a, dst_a, ew_a, rows_a, sem_a)

    plsc.subcore_barrier()
    pltpu.sync_copy(acc.at[pl.ds(base, ROWS_PER_TILE)],
                    out_hbm.at[c].at[pl.ds(base, ROWS_PER_TILE)])


def _dinv_from(degp_ref):
    deg = degp_ref[0][:, 0:1] + degp_ref[1][:, 0:1] + 1.0
    return lax.rsqrt(deg)


def _tc_a_body(degp_ref, x_ref, w1_ref, y1_ref):
    dinv = _dinv_from(degp_ref)
    xw = jnp.dot(x_ref[...], w1_ref[...], preferred_element_type=jnp.float32)
    y = dinv * xw
    y1_ref[...] = jnp.concatenate([y, jnp.zeros_like(y)], axis=1)


def _tc_b_body(degp_ref, p_ref, y_ref, b_ref, w_ref, out_ref):
    dinv = _dinv_from(degp_ref)
    y = y_ref[...][:, 0:HID]
    agg = p_ref[0][:, 0:HID] + p_ref[1][:, 0:HID]
    h = jnp.maximum(dinv * (agg + y) + b_ref[...], 0.0)
    y2 = dinv * jnp.dot(h, w_ref[...], preferred_element_type=jnp.float32)
    out_ref[...] = jnp.concatenate([y2, jnp.zeros_like(y2)], axis=1)


def _tc_c_body(degp_ref, q_ref, y_ref, b_ref, wfc_ref, bfc_ref, out_ref):
    dinv = _dinv_from(degp_ref)
    y = y_ref[...][:, 0:HID]
    agg = q_ref[0][:, 0:HID] + q_ref[1][:, 0:HID]
    h = jnp.maximum(dinv * (agg + y) + b_ref[...], 0.0)
    out_ref[...] = jnp.dot(h, wfc_ref[...],
                           preferred_element_type=jnp.float32) + bfc_ref[...]


_BN = 2000
_GRID = N_NODES // _BN


def _row_spec(width):
    return pl.BlockSpec((_BN, width), lambda i: (i, 0))


def _pair_spec(width):
    return pl.BlockSpec((NC, _BN, width), lambda i: (0, i, 0))


def _full_spec(shape):
    return pl.BlockSpec(shape, lambda i: tuple(0 for _ in shape))


def kernel(x, edge_index, edge_attr, W1, b1, W2, b2, Wfc, bfc):
    src = edge_index[0]
    dst = edge_index[1]

    degp = _deg_kernel(dst, edge_attr)

    y1 = pl.pallas_call(
        _tc_a_body,
        grid=(_GRID,),
        in_specs=[_pair_spec(128), _row_spec(128), _full_spec(W1.shape)],
        out_specs=_row_spec(2 * HID),
        out_shape=jax.ShapeDtypeStruct((N_NODES, 2 * HID), jnp.float32),
    )(degp, x, W1)

    p = _agg_kernel(y1, src, dst, edge_attr)

    y2 = pl.pallas_call(
        _tc_b_body,
        grid=(_GRID,),
        in_specs=[_pair_spec(128), _pair_spec(128), _row_spec(2 * HID),
                  _full_spec((1, HID)), _full_spec(W2.shape)],
        out_specs=_row_spec(2 * HID),
        out_shape=jax.ShapeDtypeStruct((N_NODES, 2 * HID), jnp.float32),
    )(degp, p, y1, b1.reshape(1, HID), W2)

    q = _agg_kernel(y2, src, dst, edge_attr)

    out = pl.pallas_call(
        _tc_c_body,
        grid=(_GRID,),
        in_specs=[_pair_spec(128), _pair_spec(128), _row_spec(2 * HID),
                  _full_spec((1, HID)), _full_spec(Wfc.shape),
                  _full_spec((1, Wfc.shape[1]))],
        out_specs=_row_spec(Wfc.shape[1]),
        out_shape=jax.ShapeDtypeStruct((N_NODES, Wfc.shape[1]), jnp.float32),
    )(degp, q, y2, b2.reshape(1, HID), Wfc, bfc.reshape(1, -1))

    return out


# speedup vs baseline: 16.5409x; 1.4992x over previous
import functools

import jax
import jax.numpy as jnp
from jax import lax
from jax.experimental import pallas as pl
from jax.experimental.pallas import tpu as pltpu
from jax.experimental.pallas import tpu_sc as plsc

N_NODES = 10000
N_PAD = 10240
N_EDGES = 320000
HID = 64
DEGW = 16
NC = 2
NS = 16
EDGES_PER_TILE = N_EDGES // (NC * NS)
CHUNK = 80
NCHUNKS = EDGES_PER_TILE // CHUNK
ROWS_PER_TILE = N_PAD // NS

_mesh = plsc.VectorSubcoreMesh(core_axis_name="c", subcore_axis_name="s")


@functools.partial(
    pl.kernel,
    out_type=jax.ShapeDtypeStruct((NC, N_PAD, 128), jnp.float32),
    mesh=_mesh,
    scratch_types=[
        pltpu.VMEM((CHUNK,), jnp.int32),
        pltpu.VMEM((CHUNK + 16,), jnp.float32),
        pltpu.VMEM((CHUNK, 128), jnp.float32),
        pltpu.VMEM_SHARED((N_PAD, 128), jnp.float32),
    ],
)
def _deg_kernel(dst_hbm, ew_hbm, out_hbm, dst_v, ew_v, val_v, acc):
    c = lax.axis_index("c")
    s = lax.axis_index("s")
    wid = s * NC + c
    base = s * ROWS_PER_TILE
    zeros_f = jnp.zeros((16,), jnp.float32)
    lane0 = lax.iota(jnp.int32, 16) == 0

    def zero_val(e, carry):
        for j in range(8):
            val_v[e, pl.ds(j * 16, 16)] = zeros_f
        return carry

    lax.fori_loop(0, CHUNK, zero_val, 0)
    for k in range(ROWS_PER_TILE // CHUNK):
        pltpu.sync_copy(val_v, acc.at[pl.ds(base + k * CHUNK, CHUNK)])
    plsc.subcore_barrier()

    def body(i, carry):
        off = wid * EDGES_PER_TILE + i * CHUNK
        pltpu.sync_copy(dst_hbm.at[pl.ds(off, CHUNK)], dst_v)
        pltpu.sync_copy(ew_hbm.at[pl.ds(off, CHUNK)], ew_v.at[pl.ds(0, CHUNK)])

        def fill_body(e, carry2):
            w = ew_v[pl.ds(e, 16)][0]
            val_v[e, pl.ds(0, 16)] = jnp.where(lane0, zeros_f + w, zeros_f)
            return carry2

        lax.fori_loop(0, CHUNK, fill_body, 0)
        pltpu.sync_copy(val_v, acc.at[dst_v], add=True)
        return carry

    lax.fori_loop(0, NCHUNKS, body, 0)
    plsc.subcore_barrier()
    pltpu.sync_copy(acc.at[pl.ds(base, ROWS_PER_TILE)],
                    out_hbm.at[c].at[pl.ds(base, ROWS_PER_TILE)])


@functools.partial(
    pl.kernel,
    out_type=jax.ShapeDtypeStruct((NC, N_PAD, 128), jnp.float32),
    mesh=_mesh,
    scratch_types=[
        pltpu.VMEM((CHUNK,), jnp.int32),
        pltpu.VMEM((CHUNK,), jnp.int32),
        pltpu.VMEM((CHUNK,), jnp.int32),
        pltpu.VMEM((CHUNK,), jnp.int32),
        pltpu.VMEM((CHUNK + 16,), jnp.float32),
        pltpu.VMEM((CHUNK + 16,), jnp.float32),
        pltpu.VMEM((CHUNK, 128), jnp.float32),
        pltpu.VMEM((CHUNK, 128), jnp.float32),
        pltpu.VMEM((CHUNK, 128), jnp.float32),
        pltpu.VMEM((CHUNK, 128), jnp.float32),
        pltpu.VMEM_SHARED((N_PAD, 128), jnp.float32),
    ] + [pltpu.SemaphoreType.DMA] * 10,
)
def _agg_kernel(y_hbm, src_hbm, dst_hbm, ew_hbm, out_hbm,
                src_a, src_b, dst_a, dst_b, ew_a, ew_b,
                rows_a, rows_b, val_a, val_b, acc,
                sem_ga, sem_gb, sem_sa, sem_sb,
                sem_ia0, sem_ia2, sem_ib0, sem_ib2, sem_da, sem_db):
    c = lax.axis_index("c")
    s = lax.axis_index("s")
    wid = s * NC + c
    base = s * ROWS_PER_TILE
    ebase = wid * EDGES_PER_TILE
    zeros_f = jnp.zeros((16,), jnp.float32)

    def zero_plane(v):
        def zero_val(e, carry):
            for j in range(8):
                v[e, pl.ds(j * 16, 16)] = zeros_f
            return carry
        lax.fori_loop(0, CHUNK, zero_val, 0)

    zero_plane(val_a)
    zero_plane(val_b)
    for k in range(ROWS_PER_TILE // CHUNK):
        pltpu.sync_copy(val_a, acc.at[pl.ds(base + k * CHUNK, CHUNK)])
    plsc.subcore_barrier()

    def idx_start(i, src_v, ew_v, s0, s2):
        off = ebase + i * CHUNK
        pltpu.make_async_copy(src_hbm.at[pl.ds(off, CHUNK)], src_v, s0).start()
        pltpu.make_async_copy(ew_hbm.at[pl.ds(off, CHUNK)],
                              ew_v.at[pl.ds(0, CHUNK)], s2).start()

    def idx_wait(i, src_v, ew_v, s0, s2):
        off = ebase + i * CHUNK
        pltpu.make_async_copy(src_hbm.at[pl.ds(off, CHUNK)], src_v, s0).wait()
        pltpu.make_async_copy(ew_hbm.at[pl.ds(off, CHUNK)],
                              ew_v.at[pl.ds(0, CHUNK)], s2).wait()

    def dst_start(i, dst_v, sd):
        off = ebase + i * CHUNK
        pltpu.make_async_copy(dst_hbm.at[pl.ds(off, CHUNK)], dst_v, sd).start()

    def dst_wait(i, dst_v, sd):
        off = ebase + i * CHUNK
        pltpu.make_async_copy(dst_hbm.at[pl.ds(off, CHUNK)], dst_v, sd).wait()

    def scat_start(val_v, dst_v, sem_s):
        pltpu.async_copy(val_v, acc.at[dst_v], sem_s, add=True)

    def scat_wait(val_v, dst_v, sem_s):
        pltpu.make_async_copy(val_v, acc.at[dst_v], sem_s).wait()

    def scale(ew_v, rows_v, val_v):
        def scale_body(e, carry2):
            w = zeros_f + ew_v[pl.ds(e, 16)][0]
            for j in range(HID // 16):
                val_v[e, pl.ds(j * 16, 16)] = rows_v[e, pl.ds(j * 16, 16)] * w
            return carry2
        lax.fori_loop(0, CHUNK, scale_body, 0)

    idx_start(0, src_a, ew_a, sem_ia0, sem_ia2)
    idx_wait(0, src_a, ew_a, sem_ia0, sem_ia2)
    pltpu.make_async_copy(y_hbm.at[src_a], rows_a, sem_ga).start()
    pltpu.sync_copy(dst_hbm.at[pl.ds(ebase, CHUNK)], dst_a)
    pltpu.sync_copy(dst_hbm.at[pl.ds(ebase + CHUNK, CHUNK)], dst_b)
    scat_start(val_a, dst_a, sem_sa)
    scat_start(val_b, dst_b, sem_sb)
    idx_start(1, src_b, ew_b, sem_ib0, sem_ib2)

    def src_start(i, src_v, s0):
        off = ebase + i * CHUNK
        pltpu.make_async_copy(src_hbm.at[pl.ds(off, CHUNK)], src_v, s0).start()

    def ew_start(i, ew_v, s2):
        off = ebase + i * CHUNK
        pltpu.make_async_copy(ew_hbm.at[pl.ds(off, CHUNK)],
                              ew_v.at[pl.ds(0, CHUNK)], s2).start()

    def half(i_cur, i_nxt,
             src_c, ew_c, dst_c, rows_c, val_c, sem_gc, sem_sc, sem_ic0,
             sem_ic2, sem_dc):
        pltpu.make_async_copy(y_hbm.at[src_c], rows_c, sem_gc).wait()
        src_start(i_nxt, src_c, sem_ic0)
        scat_wait(val_c, dst_c, sem_sc)
        dst_start(i_cur, dst_c, sem_dc)
        scale(ew_c, rows_c, val_c)
        ew_start(i_nxt, ew_c, sem_ic2)
        dst_wait(i_cur, dst_c, sem_dc)
        scat_start(val_c, dst_c, sem_sc)

    def pair_body(j, carry):
        idx_wait(2 * j + 1, src_b, ew_b, sem_ib0, sem_ib2)
        pltpu.make_async_copy(y_hbm.at[src_b], rows_b, sem_gb).start()
        half(2 * j, 2 * j + 2,
             src_a, ew_a, dst_a, rows_a, val_a, sem_ga, sem_sa, sem_ia0,
             sem_ia2, sem_da)
        idx_wait(2 * j + 2, src_a, ew_a, sem_ia0, sem_ia2)
        pltpu.make_async_copy(y_hbm.at[src_a], rows_a, sem_ga).start()
        half(2 * j + 1, 2 * j + 3,
             src_b, ew_b, dst_b, rows_b, val_b, sem_gb, sem_sb, sem_ib0,
             sem_ib2, sem_db)
        return carry

    lax.fori_loop(0, 61, pair_body, 0)
    idx_wait(123, src_b, ew_b, sem_ib0, sem_ib2)
    pltpu.make_async_copy(y_hbm.at[src_b], rows_b, sem_gb).start()
    half(122, 124,
         src_a, ew_a, dst_a, rows_a, val_a, sem_ga, sem_sa, sem_ia0,
         sem_ia2, sem_da)
    pltpu.make_async_copy(y_hbm.at[src_b], rows_b, sem_gb).wait()
    scat_wait(val_b, dst_b, sem_sb)
    dst_start(123, dst_b, sem_db)
    scale(ew_b, rows_b, val_b)
    dst_wait(123, dst_b, sem_db)
    scat_start(val_b, dst_b, sem_sb)
    idx_wait(124, src_a, ew_a, sem_ia0, sem_ia2)
    pltpu.make_async_copy(y_hbm.at[src_a], rows_a, sem_ga).start()
    pltpu.make_async_copy(y_hbm.at[src_a], rows_a, sem_ga).wait()
    scat_wait(val_a, dst_a, sem_sa)
    dst_start(124, dst_a, sem_da)
    scale(ew_a, rows_a, val_a)
    dst_wait(124, dst_a, sem_da)
    scat_start(val_a, dst_a, sem_sa)
    scat_wait(val_a, dst_a, sem_sa)
    scat_wait(val_b, dst_b, sem_sb)

    plsc.subcore_barrier()
    pltpu.sync_copy(acc.at[pl.ds(base, ROWS_PER_TILE)],
                    out_hbm.at[c].at[pl.ds(base, ROWS_PER_TILE)])


def _dinv_from(degp_ref):
    deg = degp_ref[0][:, 0:1] + degp_ref[1][:, 0:1] + 1.0
    return lax.rsqrt(deg)


def _tc_a_body(degp_ref, x_ref, w1_ref, y1_ref):
    dinv = _dinv_from(degp_ref)
    xw = jnp.dot(x_ref[...], w1_ref[...], preferred_element_type=jnp.float32)
    y = dinv * xw
    y1_ref[...] = jnp.concatenate([y, jnp.zeros_like(y)], axis=1)


def _tc_b_body(degp_ref, p_ref, y_ref, b_ref, w_ref, out_ref):
    dinv = _dinv_from(degp_ref)
    y = y_ref[...][:, 0:HID]
    agg = p_ref[0][:, 0:HID] + p_ref[1][:, 0:HID]
    h = jnp.maximum(dinv * (agg + y) + b_ref[...], 0.0)
    y2 = dinv * jnp.dot(h, w_ref[...], preferred_element_type=jnp.float32)
    out_ref[...] = jnp.concatenate([y2, jnp.zeros_like(y2)], axis=1)


def _tc_c_body(degp_ref, q_ref, y_ref, b_ref, wfc_ref, bfc_ref, out_ref):
    dinv = _dinv_from(degp_ref)
    y = y_ref[...][:, 0:HID]
    agg = q_ref[0][:, 0:HID] + q_ref[1][:, 0:HID]
    h = jnp.maximum(dinv * (agg + y) + b_ref[...], 0.0)
    out_ref[...] = jnp.dot(h, wfc_ref[...],
                           preferred_element_type=jnp.float32) + bfc_ref[...]


_BN = 2000
_GRID = N_NODES // _BN


def _row_spec(width):
    return pl.BlockSpec((_BN, width), lambda i: (i, 0))


def _pair_spec(width):
    return pl.BlockSpec((NC, _BN, width), lambda i: (0, i, 0))


def _full_spec(shape):
    return pl.BlockSpec(shape, lambda i: tuple(0 for _ in shape))


def kernel(x, edge_index, edge_attr, W1, b1, W2, b2, Wfc, bfc):
    src = edge_index[0]
    dst = edge_index[1]

    degp = _deg_kernel(dst, edge_attr)

    y1 = pl.pallas_call(
        _tc_a_body,
        grid=(_GRID,),
        in_specs=[_pair_spec(128), _row_spec(128), _full_spec(W1.shape)],
        out_specs=_row_spec(2 * HID),
        out_shape=jax.ShapeDtypeStruct((N_NODES, 2 * HID), jnp.float32),
    )(degp, x, W1)

    p = _agg_kernel(y1, src, dst, edge_attr)

    y2 = pl.pallas_call(
        _tc_b_body,
        grid=(_GRID,),
        in_specs=[_pair_spec(128), _pair_spec(128), _row_spec(2 * HID),
                  _full_spec((1, HID)), _full_spec(W2.shape)],
        out_specs=_row_spec(2 * HID),
        out_shape=jax.ShapeDtypeStruct((N_NODES, 2 * HID), jnp.float32),
    )(degp, p, y1, b1.reshape(1, HID), W2)

    q = _agg_kernel(y2, src, dst, edge_attr)

    out = pl.pallas_call(
        _tc_c_body,
        grid=(_GRID,),
        in_specs=[_pair_spec(128), _pair_spec(128), _row_spec(2 * HID),
                  _full_spec((1, HID)), _full_spec(Wfc.shape),
                  _full_spec((1, Wfc.shape[1]))],
        out_specs=_row_spec(Wfc.shape[1]),
        out_shape=jax.ShapeDtypeStruct((N_NODES, Wfc.shape[1]), jnp.float32),
    )(degp, q, y2, b2.reshape(1, HID), Wfc, bfc.reshape(1, -1))

    return out


# speedup vs baseline: 22.0808x; 1.3349x over previous
import functools

import jax
import jax.numpy as jnp
from jax import lax
from jax.experimental import pallas as pl
from jax.experimental.pallas import tpu as pltpu
from jax.experimental.pallas import tpu_sc as plsc

N_NODES = 10000
N_PAD = 10240
N_EDGES = 320000
HID = 64
DEGW = 16
NC = 2
NS = 16
EDGES_PER_TILE = N_EDGES // (NC * NS)
CHUNK = 80
NCHUNKS = EDGES_PER_TILE // CHUNK
ROWS_PER_TILE = N_PAD // NS

_mesh = plsc.VectorSubcoreMesh(core_axis_name="c", subcore_axis_name="s")


@functools.partial(
    pl.kernel,
    out_type=jax.ShapeDtypeStruct((NC, N_PAD, 128), jnp.float32),
    mesh=_mesh,
    scratch_types=[
        pltpu.VMEM((CHUNK,), jnp.int32),
        pltpu.VMEM((CHUNK,), jnp.int32),
        pltpu.VMEM((CHUNK + 16,), jnp.float32),
        pltpu.VMEM((CHUNK + 16,), jnp.float32),
        pltpu.VMEM((CHUNK, 128), jnp.float32),
        pltpu.VMEM((CHUNK, 128), jnp.float32),
        pltpu.VMEM_SHARED((N_PAD, 128), jnp.float32),
    ] + [pltpu.SemaphoreType.DMA] * 6,
)
def _deg_kernel(dst_hbm, ew_hbm, out_hbm,
                dst_a, dst_b, ew_a, ew_b, val_a, val_b, acc,
                sem_sa, sem_sb, sem_ea, sem_eb, sem_da, sem_db):
    c = lax.axis_index("c")
    s = lax.axis_index("s")
    wid = s * NC + c
    base = s * ROWS_PER_TILE
    ebase = wid * EDGES_PER_TILE
    zeros_f = jnp.zeros((16,), jnp.float32)
    lane0 = lax.iota(jnp.int32, 16) == 0

    def zero_plane(v):
        def zero_val(e, carry):
            for j in range(8):
                v[e, pl.ds(j * 16, 16)] = zeros_f
            return carry
        lax.fori_loop(0, CHUNK, zero_val, 0)

    zero_plane(val_a)
    zero_plane(val_b)
    for k in range(ROWS_PER_TILE // CHUNK):
        pltpu.sync_copy(val_a, acc.at[pl.ds(base + k * CHUNK, CHUNK)])
    plsc.subcore_barrier()

    def ew_start(i, ew_v, se):
        off = ebase + i * CHUNK
        pltpu.make_async_copy(ew_hbm.at[pl.ds(off, CHUNK)],
                              ew_v.at[pl.ds(0, CHUNK)], se).start()

    def ew_wait(i, ew_v, se):
        off = ebase + i * CHUNK
        pltpu.make_async_copy(ew_hbm.at[pl.ds(off, CHUNK)],
                              ew_v.at[pl.ds(0, CHUNK)], se).wait()

    def dst_start(i, dst_v, sd):
        off = ebase + i * CHUNK
        pltpu.make_async_copy(dst_hbm.at[pl.ds(off, CHUNK)], dst_v, sd).start()

    def dst_wait(i, dst_v, sd):
        off = ebase + i * CHUNK
        pltpu.make_async_copy(dst_hbm.at[pl.ds(off, CHUNK)], dst_v, sd).wait()

    def fill(ew_v, val_v):
        def fill_body(e, carry2):
            w = ew_v[pl.ds(e, 16)][0]
            val_v[e, pl.ds(0, 16)] = jnp.where(lane0, zeros_f + w, zeros_f)
            return carry2
        lax.fori_loop(0, CHUNK, fill_body, 0)

    def half(i_cur, i_nxt, ew_c, dst_c, val_c, sem_ec, sem_sc, sem_dc):
        ew_wait(i_cur, ew_c, sem_ec)
        pltpu.make_async_copy(val_c, acc.at[dst_c], sem_sc).wait()
        dst_start(i_cur, dst_c, sem_dc)
        fill(ew_c, val_c)
        if i_nxt is not None:
            ew_start(i_nxt, ew_c, sem_ec)
        dst_wait(i_cur, dst_c, sem_dc)
        pltpu.async_copy(val_c, acc.at[dst_c], sem_sc, add=True)

    ew_start(0, ew_a, sem_ea)
    ew_start(1, ew_b, sem_eb)
    pltpu.sync_copy(dst_hbm.at[pl.ds(ebase, CHUNK)], dst_a)
    pltpu.sync_copy(dst_hbm.at[pl.ds(ebase + CHUNK, CHUNK)], dst_b)
    pltpu.async_copy(val_a, acc.at[dst_a], sem_sa, add=True)
    pltpu.async_copy(val_b, acc.at[dst_b], sem_sb, add=True)

    def pair_body(j, carry):
        half(2 * j, 2 * j + 2, ew_a, dst_a, val_a, sem_ea, sem_sa, sem_da)
        half(2 * j + 1, 2 * j + 3, ew_b, dst_b, val_b, sem_eb, sem_sb, sem_db)
        return carry

    lax.fori_loop(0, 61, pair_body, 0)
    half(122, 124, ew_a, dst_a, val_a, sem_ea, sem_sa, sem_da)
    half(123, None, ew_b, dst_b, val_b, sem_eb, sem_sb, sem_db)
    half(124, None, ew_a, dst_a, val_a, sem_ea, sem_sa, sem_da)
    pltpu.make_async_copy(val_a, acc.at[dst_a], sem_sa).wait()
    pltpu.make_async_copy(val_b, acc.at[dst_b], sem_sb).wait()

    plsc.subcore_barrier()
    pltpu.sync_copy(acc.at[pl.ds(base, ROWS_PER_TILE)],
                    out_hbm.at[c].at[pl.ds(base, ROWS_PER_TILE)])


@functools.partial(
    pl.kernel,
    out_type=jax.ShapeDtypeStruct((NC, N_PAD, 128), jnp.float32),
    mesh=_mesh,
    scratch_types=[
        pltpu.VMEM((CHUNK,), jnp.int32),
        pltpu.VMEM((CHUNK,), jnp.int32),
        pltpu.VMEM((CHUNK,), jnp.int32),
        pltpu.VMEM((CHUNK,), jnp.int32),
        pltpu.VMEM((CHUNK + 16,), jnp.float32),
        pltpu.VMEM((CHUNK + 16,), jnp.float32),
        pltpu.VMEM((CHUNK, 128), jnp.float32),
        pltpu.VMEM((CHUNK, 128), jnp.float32),
        pltpu.VMEM((CHUNK, 128), jnp.float32),
        pltpu.VMEM((CHUNK, 128), jnp.float32),
        pltpu.VMEM_SHARED((N_PAD, 128), jnp.float32),
    ] + [pltpu.SemaphoreType.DMA] * 10,
)
def _agg_kernel(y_hbm, src_hbm, dst_hbm, ew_hbm, out_hbm,
                src_a, src_b, dst_a, dst_b, ew_a, ew_b,
                rows_a, rows_b, val_a, val_b, acc,
                sem_ga, sem_gb, sem_sa, sem_sb,
                sem_ia0, sem_ia2, sem_ib0, sem_ib2, sem_da, sem_db):
    c = lax.axis_index("c")
    s = lax.axis_index("s")
    wid = s * NC + c
    base = s * ROWS_PER_TILE
    ebase = wid * EDGES_PER_TILE
    zeros_f = jnp.zeros((16,), jnp.float32)

    def zero_plane(v):
        def zero_val(e, carry):
            for j in range(8):
                v[e, pl.ds(j * 16, 16)] = zeros_f
            return carry
        lax.fori_loop(0, CHUNK, zero_val, 0)

    zero_plane(val_a)
    zero_plane(val_b)
    for k in range(ROWS_PER_TILE // CHUNK):
        pltpu.sync_copy(val_a, acc.at[pl.ds(base + k * CHUNK, CHUNK)])
    plsc.subcore_barrier()

    def idx_start(i, src_v, ew_v, s0, s2):
        off = ebase + i * CHUNK
        pltpu.make_async_copy(src_hbm.at[pl.ds(off, CHUNK)], src_v, s0).start()
        pltpu.make_async_copy(ew_hbm.at[pl.ds(off, CHUNK)],
                              ew_v.at[pl.ds(0, CHUNK)], s2).start()

    def idx_wait(i, src_v, ew_v, s0, s2):
        off = ebase + i * CHUNK
        pltpu.make_async_copy(src_hbm.at[pl.ds(off, CHUNK)], src_v, s0).wait()
        pltpu.make_async_copy(ew_hbm.at[pl.ds(off, CHUNK)],
                              ew_v.at[pl.ds(0, CHUNK)], s2).wait()

    def dst_start(i, dst_v, sd):
        off = ebase + i * CHUNK
        pltpu.make_async_copy(dst_hbm.at[pl.ds(off, CHUNK)], dst_v, sd).start()

    def dst_wait(i, dst_v, sd):
        off = ebase + i * CHUNK
        pltpu.make_async_copy(dst_hbm.at[pl.ds(off, CHUNK)], dst_v, sd).wait()

    def scat_start(val_v, dst_v, sem_s):
        pltpu.async_copy(val_v, acc.at[dst_v], sem_s, add=True)

    def scat_wait(val_v, dst_v, sem_s):
        pltpu.make_async_copy(val_v, acc.at[dst_v], sem_s).wait()

    def scale(ew_v, rows_v, val_v):
        def scale_body(e, carry2):
            w = zeros_f + ew_v[pl.ds(e, 16)][0]
            for j in range(HID // 16):
                val_v[e, pl.ds(j * 16, 16)] = rows_v[e, pl.ds(j * 16, 16)] * w
            return carry2
        lax.fori_loop(0, CHUNK, scale_body, 0)

    idx_start(0, src_a, ew_a, sem_ia0, sem_ia2)
    idx_wait(0, src_a, ew_a, sem_ia0, sem_ia2)
    pltpu.make_async_copy(y_hbm.at[src_a], rows_a, sem_ga).start()
    pltpu.sync_copy(dst_hbm.at[pl.ds(ebase, CHUNK)], dst_a)
    pltpu.sync_copy(dst_hbm.at[pl.ds(ebase + CHUNK, CHUNK)], dst_b)
    scat_start(val_a, dst_a, sem_sa)
    scat_start(val_b, dst_b, sem_sb)
    idx_start(1, src_b, ew_b, sem_ib0, sem_ib2)

    def src_start(i, src_v, s0):
        off = ebase + i * CHUNK
        pltpu.make_async_copy(src_hbm.at[pl.ds(off, CHUNK)], src_v, s0).start()

    def ew_start(i, ew_v, s2):
        off = ebase + i * CHUNK
        pltpu.make_async_copy(ew_hbm.at[pl.ds(off, CHUNK)],
                              ew_v.at[pl.ds(0, CHUNK)], s2).start()

    def half(i_cur, i_nxt,
             src_c, ew_c, dst_c, rows_c, val_c, sem_gc, sem_sc, sem_ic0,
             sem_ic2, sem_dc):
        pltpu.make_async_copy(y_hbm.at[src_c], rows_c, sem_gc).wait()
        src_start(i_nxt, src_c, sem_ic0)
        scat_wait(val_c, dst_c, sem_sc)
        dst_start(i_cur, dst_c, sem_dc)
        scale(ew_c, rows_c, val_c)
        ew_start(i_nxt, ew_c, sem_ic2)
        dst_wait(i_cur, dst_c, sem_dc)
        scat_start(val_c, dst_c, sem_sc)

    def pair_body(j, carry):
        idx_wait(2 * j + 1, src_b, ew_b, sem_ib0, sem_ib2)
        pltpu.make_async_copy(y_hbm.at[src_b], rows_b, sem_gb).start()
        half(2 * j, 2 * j + 2,
             src_a, ew_a, dst_a, rows_a, val_a, sem_ga, sem_sa, sem_ia0,
             sem_ia2, sem_da)
        idx_wait(2 * j + 2, src_a, ew_a, sem_ia0, sem_ia2)
        pltpu.make_async_copy(y_hbm.at[src_a], rows_a, sem_ga).start()
        half(2 * j + 1, 2 * j + 3,
             src_b, ew_b, dst_b, rows_b, val_b, sem_gb, sem_sb, sem_ib0,
             sem_ib2, sem_db)
        return carry

    lax.fori_loop(0, 61, pair_body, 0)
    idx_wait(123, src_b, ew_b, sem_ib0, sem_ib2)
    pltpu.make_async_copy(y_hbm.at[src_b], rows_b, sem_gb).start()
    half(122, 124,
         src_a, ew_a, dst_a, rows_a, val_a, sem_ga, sem_sa, sem_ia0,
         sem_ia2, sem_da)
    pltpu.make_async_copy(y_hbm.at[src_b], rows_b, sem_gb).wait()
    scat_wait(val_b, dst_b, sem_sb)
    dst_start(123, dst_b, sem_db)
    scale(ew_b, rows_b, val_b)
    dst_wait(123, dst_b, sem_db)
    scat_start(val_b, dst_b, sem_sb)
    idx_wait(124, src_a, ew_a, sem_ia0, sem_ia2)
    pltpu.make_async_copy(y_hbm.at[src_a], rows_a, sem_ga).start()
    pltpu.make_async_copy(y_hbm.at[src_a], rows_a, sem_ga).wait()
    scat_wait(val_a, dst_a, sem_sa)
    dst_start(124, dst_a, sem_da)
    scale(ew_a, rows_a, val_a)
    dst_wait(124, dst_a, sem_da)
    scat_start(val_a, dst_a, sem_sa)
    scat_wait(val_a, dst_a, sem_sa)
    scat_wait(val_b, dst_b, sem_sb)

    plsc.subcore_barrier()
    pltpu.sync_copy(acc.at[pl.ds(base, ROWS_PER_TILE)],
                    out_hbm.at[c].at[pl.ds(base, ROWS_PER_TILE)])


def _dinv_from(degp_ref):
    deg = degp_ref[0][:, 0:1] + degp_ref[1][:, 0:1] + 1.0
    return lax.rsqrt(deg)


def _tc_a_body(degp_ref, x_ref, w1_ref, y1_ref):
    dinv = _dinv_from(degp_ref)
    xw = jnp.dot(x_ref[...], w1_ref[...], preferred_element_type=jnp.float32)
    y = dinv * xw
    y1_ref[...] = jnp.concatenate([y, jnp.zeros_like(y)], axis=1)


def _tc_b_body(degp_ref, p_ref, y_ref, b_ref, w_ref, out_ref):
    dinv = _dinv_from(degp_ref)
    y = y_ref[...][:, 0:HID]
    agg = p_ref[0][:, 0:HID] + p_ref[1][:, 0:HID]
    h = jnp.maximum(dinv * (agg + y) + b_ref[...], 0.0)
    y2 = dinv * jnp.dot(h, w_ref[...], preferred_element_type=jnp.float32)
    out_ref[...] = jnp.concatenate([y2, jnp.zeros_like(y2)], axis=1)


def _tc_c_body(degp_ref, q_ref, y_ref, b_ref, wfc_ref, bfc_ref, out_ref):
    dinv = _dinv_from(degp_ref)
    y = y_ref[...][:, 0:HID]
    agg = q_ref[0][:, 0:HID] + q_ref[1][:, 0:HID]
    h = jnp.maximum(dinv * (agg + y) + b_ref[...], 0.0)
    out_ref[...] = jnp.dot(h, wfc_ref[...],
                           preferred_element_type=jnp.float32) + bfc_ref[...]


_BN = 2000
_GRID = N_NODES // _BN


def _row_spec(width):
    return pl.BlockSpec((_BN, width), lambda i: (i, 0))


def _pair_spec(width):
    return pl.BlockSpec((NC, _BN, width), lambda i: (0, i, 0))


def _full_spec(shape):
    return pl.BlockSpec(shape, lambda i: tuple(0 for _ in shape))


def kernel(x, edge_index, edge_attr, W1, b1, W2, b2, Wfc, bfc):
    src = edge_index[0]
    dst = edge_index[1]

    degp = _deg_kernel(dst, edge_attr)

    y1 = pl.pallas_call(
        _tc_a_body,
        grid=(_GRID,),
        in_specs=[_pair_spec(128), _row_spec(128), _full_spec(W1.shape)],
        out_specs=_row_spec(2 * HID),
        out_shape=jax.ShapeDtypeStruct((N_NODES, 2 * HID), jnp.float32),
    )(degp, x, W1)

    p = _agg_kernel(y1, src, dst, edge_attr)

    y2 = pl.pallas_call(
        _tc_b_body,
        grid=(_GRID,),
        in_specs=[_pair_spec(128), _pair_spec(128), _row_spec(2 * HID),
                  _full_spec((1, HID)), _full_spec(W2.shape)],
        out_specs=_row_spec(2 * HID),
        out_shape=jax.ShapeDtypeStruct((N_NODES, 2 * HID), jnp.float32),
    )(degp, p, y1, b1.reshape(1, HID), W2)

    q = _agg_kernel(y2, src, dst, edge_attr)

    out = pl.pallas_call(
        _tc_c_body,
        grid=(_GRID,),
        in_specs=[_pair_spec(128), _pair_spec(128), _row_spec(2 * HID),
                  _full_spec((1, HID)), _full_spec(Wfc.shape),
                  _full_spec((1, Wfc.shape[1]))],
        out_specs=_row_spec(Wfc.shape[1]),
        out_shape=jax.ShapeDtypeStruct((N_NODES, Wfc.shape[1]), jnp.float32),
    )(degp, q, y2, b2.reshape(1, HID), Wfc, bfc.reshape(1, -1))

    return out


# speedup vs baseline: 23.8127x; 1.0784x over previous
import functools

import jax
import jax.numpy as jnp
from jax import lax
from jax.experimental import pallas as pl
from jax.experimental.pallas import tpu as pltpu
from jax.experimental.pallas import tpu_sc as plsc

N_NODES = 10000
N_PAD = 10240
N_EDGES = 320000
HID = 64
DEGW = 16
NC = 2
NS = 16
EDGES_PER_TILE = N_EDGES // (NC * NS)
CHUNK = 80
NCHUNKS = EDGES_PER_TILE // CHUNK
ROWS_PER_TILE = N_PAD // NS

_mesh = plsc.VectorSubcoreMesh(core_axis_name="c", subcore_axis_name="s")


@functools.partial(
    pl.kernel,
    out_type=jax.ShapeDtypeStruct((NC, N_PAD, 128), jnp.float32),
    mesh=_mesh,
    scratch_types=[
        pltpu.VMEM((CHUNK,), jnp.int32),
        pltpu.VMEM((CHUNK,), jnp.int32),
        pltpu.VMEM((CHUNK + 16,), jnp.float32),
        pltpu.VMEM((CHUNK + 16,), jnp.float32),
        pltpu.VMEM((CHUNK, 128), jnp.float32),
        pltpu.VMEM((CHUNK, 128), jnp.float32),
        pltpu.VMEM_SHARED((N_PAD, 128), jnp.float32),
    ] + [pltpu.SemaphoreType.DMA] * 6,
)
def _deg_kernel(dst_hbm, ew_hbm, out_hbm,
                dst_a, dst_b, ew_a, ew_b, val_a, val_b, acc,
                sem_sa, sem_sb, sem_ea, sem_eb, sem_da, sem_db):
    c = lax.axis_index("c")
    s = lax.axis_index("s")
    wid = s * NC + c
    base = s * ROWS_PER_TILE
    ebase = wid * EDGES_PER_TILE
    zeros_f = jnp.zeros((16,), jnp.float32)
    lane0 = lax.iota(jnp.int32, 16) == 0

    def zero_plane(v):
        def zero_val(e, carry):
            for j in range(8):
                v[e, pl.ds(j * 16, 16)] = zeros_f
            return carry
        lax.fori_loop(0, CHUNK, zero_val, 0)

    zero_plane(val_a)
    zero_plane(val_b)
    for k in range(ROWS_PER_TILE // CHUNK):
        pltpu.sync_copy(val_a, acc.at[pl.ds(base + k * CHUNK, CHUNK)])
    plsc.subcore_barrier()

    def ew_start(i, ew_v, se):
        off = ebase + i * CHUNK
        pltpu.make_async_copy(ew_hbm.at[pl.ds(off, CHUNK)],
                              ew_v.at[pl.ds(0, CHUNK)], se).start()

    def ew_wait(i, ew_v, se):
        off = ebase + i * CHUNK
        pltpu.make_async_copy(ew_hbm.at[pl.ds(off, CHUNK)],
                              ew_v.at[pl.ds(0, CHUNK)], se).wait()

    def dst_start(i, dst_v, sd):
        off = ebase + i * CHUNK
        pltpu.make_async_copy(dst_hbm.at[pl.ds(off, CHUNK)], dst_v, sd).start()

    def dst_wait(i, dst_v, sd):
        off = ebase + i * CHUNK
        pltpu.make_async_copy(dst_hbm.at[pl.ds(off, CHUNK)], dst_v, sd).wait()

    def fill(ew_v, val_v):
        def fill_body(e2, carry2):
            e = 2 * e2
            w0 = ew_v[pl.ds(e, 16)][0]
            w1 = ew_v[pl.ds(e + 1, 16)][0]
            val_v[e, pl.ds(0, 16)] = jnp.where(lane0, zeros_f + w0, zeros_f)
            val_v[e + 1, pl.ds(0, 16)] = jnp.where(lane0, zeros_f + w1,
                                                   zeros_f)
            return carry2
        lax.fori_loop(0, CHUNK // 2, fill_body, 0)

    def half(i_cur, i_nxt, ew_c, dst_c, val_c, sem_ec, sem_sc, sem_dc):
        ew_wait(i_cur, ew_c, sem_ec)
        pltpu.make_async_copy(val_c, acc.at[dst_c], sem_sc).wait()
        dst_start(i_cur, dst_c, sem_dc)
        fill(ew_c, val_c)
        if i_nxt is not None:
            ew_start(i_nxt, ew_c, sem_ec)
        dst_wait(i_cur, dst_c, sem_dc)
        pltpu.async_copy(val_c, acc.at[dst_c], sem_sc, add=True)

    ew_start(0, ew_a, sem_ea)
    ew_start(1, ew_b, sem_eb)
    pltpu.sync_copy(dst_hbm.at[pl.ds(ebase, CHUNK)], dst_a)
    pltpu.sync_copy(dst_hbm.at[pl.ds(ebase + CHUNK, CHUNK)], dst_b)
    pltpu.async_copy(val_a, acc.at[dst_a], sem_sa, add=True)
    pltpu.async_copy(val_b, acc.at[dst_b], sem_sb, add=True)

    def pair_body(j, carry):
        half(2 * j, 2 * j + 2, ew_a, dst_a, val_a, sem_ea, sem_sa, sem_da)
        half(2 * j + 1, 2 * j + 3, ew_b, dst_b, val_b, sem_eb, sem_sb, sem_db)
        return carry

    lax.fori_loop(0, 61, pair_body, 0)
    half(122, 124, ew_a, dst_a, val_a, sem_ea, sem_sa, sem_da)
    half(123, None, ew_b, dst_b, val_b, sem_eb, sem_sb, sem_db)
    half(124, None, ew_a, dst_a, val_a, sem_ea, sem_sa, sem_da)
    pltpu.make_async_copy(val_a, acc.at[dst_a], sem_sa).wait()
    pltpu.make_async_copy(val_b, acc.at[dst_b], sem_sb).wait()

    plsc.subcore_barrier()
    pltpu.sync_copy(acc.at[pl.ds(base, ROWS_PER_TILE)],
                    out_hbm.at[c].at[pl.ds(base, ROWS_PER_TILE)])


@functools.partial(
    pl.kernel,
    out_type=jax.ShapeDtypeStruct((NC, N_PAD, 128), jnp.float32),
    mesh=_mesh,
    scratch_types=[
        pltpu.VMEM((CHUNK,), jnp.int32),
        pltpu.VMEM((CHUNK,), jnp.int32),
        pltpu.VMEM((CHUNK,), jnp.int32),
        pltpu.VMEM((CHUNK,), jnp.int32),
        pltpu.VMEM((CHUNK + 16,), jnp.float32),
        pltpu.VMEM((CHUNK + 16,), jnp.float32),
        pltpu.VMEM((CHUNK, 128), jnp.float32),
        pltpu.VMEM((CHUNK, 128), jnp.float32),
        pltpu.VMEM((CHUNK, 128), jnp.float32),
        pltpu.VMEM((CHUNK, 128), jnp.float32),
        pltpu.VMEM_SHARED((N_PAD, 128), jnp.float32),
    ] + [pltpu.SemaphoreType.DMA] * 10,
)
def _agg_kernel(y_hbm, src_hbm, dst_hbm, ew_hbm, out_hbm,
                src_a, src_b, dst_a, dst_b, ew_a, ew_b,
                rows_a, rows_b, val_a, val_b, acc,
                sem_ga, sem_gb, sem_sa, sem_sb,
                sem_ia0, sem_ia2, sem_ib0, sem_ib2, sem_da, sem_db):
    c = lax.axis_index("c")
    s = lax.axis_index("s")
    wid = s * NC + c
    base = s * ROWS_PER_TILE
    ebase = wid * EDGES_PER_TILE
    zeros_f = jnp.zeros((16,), jnp.float32)

    def zero_plane(v):
        def zero_val(e, carry):
            for j in range(8):
                v[e, pl.ds(j * 16, 16)] = zeros_f
            return carry
        lax.fori_loop(0, CHUNK, zero_val, 0)

    zero_plane(val_a)
    zero_plane(val_b)
    for k in range(ROWS_PER_TILE // CHUNK):
        pltpu.sync_copy(val_a, acc.at[pl.ds(base + k * CHUNK, CHUNK)])
    plsc.subcore_barrier()

    def idx_start(i, src_v, ew_v, s0, s2):
        off = ebase + i * CHUNK
        pltpu.make_async_copy(src_hbm.at[pl.ds(off, CHUNK)], src_v, s0).start()
        pltpu.make_async_copy(ew_hbm.at[pl.ds(off, CHUNK)],
                              ew_v.at[pl.ds(0, CHUNK)], s2).start()

    def idx_wait(i, src_v, ew_v, s0, s2):
        off = ebase + i * CHUNK
        pltpu.make_async_copy(src_hbm.at[pl.ds(off, CHUNK)], src_v, s0).wait()
        pltpu.make_async_copy(ew_hbm.at[pl.ds(off, CHUNK)],
                              ew_v.at[pl.ds(0, CHUNK)], s2).wait()

    def dst_start(i, dst_v, sd):
        off = ebase + i * CHUNK
        pltpu.make_async_copy(dst_hbm.at[pl.ds(off, CHUNK)], dst_v, sd).start()

    def dst_wait(i, dst_v, sd):
        off = ebase + i * CHUNK
        pltpu.make_async_copy(dst_hbm.at[pl.ds(off, CHUNK)], dst_v, sd).wait()

    def scat_start(val_v, dst_v, sem_s):
        pltpu.async_copy(val_v, acc.at[dst_v], sem_s, add=True)

    def scat_wait(val_v, dst_v, sem_s):
        pltpu.make_async_copy(val_v, acc.at[dst_v], sem_s).wait()

    def scale(ew_v, rows_v, val_v):
        def scale_body(e2, carry2):
            e = 2 * e2
            w0 = zeros_f + ew_v[pl.ds(e, 16)][0]
            w1 = zeros_f + ew_v[pl.ds(e + 1, 16)][0]
            for j in range(HID // 16):
                val_v[e, pl.ds(j * 16, 16)] = rows_v[e, pl.ds(j * 16, 16)] * w0
            for j in range(HID // 16):
                val_v[e + 1, pl.ds(j * 16, 16)] = (
                    rows_v[e + 1, pl.ds(j * 16, 16)] * w1)
            return carry2
        lax.fori_loop(0, CHUNK // 2, scale_body, 0)

    idx_start(0, src_a, ew_a, sem_ia0, sem_ia2)
    idx_wait(0, src_a, ew_a, sem_ia0, sem_ia2)
    pltpu.make_async_copy(y_hbm.at[src_a], rows_a, sem_ga).start()
    pltpu.sync_copy(dst_hbm.at[pl.ds(ebase, CHUNK)], dst_a)
    pltpu.sync_copy(dst_hbm.at[pl.ds(ebase + CHUNK, CHUNK)], dst_b)
    scat_start(val_a, dst_a, sem_sa)
    scat_start(val_b, dst_b, sem_sb)
    idx_start(1, src_b, ew_b, sem_ib0, sem_ib2)

    def src_start(i, src_v, s0):
        off = ebase + i * CHUNK
        pltpu.make_async_copy(src_hbm.at[pl.ds(off, CHUNK)], src_v, s0).start()

    def ew_start(i, ew_v, s2):
        off = ebase + i * CHUNK
        pltpu.make_async_copy(ew_hbm.at[pl.ds(off, CHUNK)],
                              ew_v.at[pl.ds(0, CHUNK)], s2).start()

    def half(i_cur, i_nxt,
             src_c, ew_c, dst_c, rows_c, val_c, sem_gc, sem_sc, sem_ic0,
             sem_ic2, sem_dc):
        pltpu.make_async_copy(y_hbm.at[src_c], rows_c, sem_gc).wait()
        src_start(i_nxt, src_c, sem_ic0)
        scat_wait(val_c, dst_c, sem_sc)
        dst_start(i_cur, dst_c, sem_dc)
        scale(ew_c, rows_c, val_c)
        ew_start(i_nxt, ew_c, sem_ic2)
        dst_wait(i_cur, dst_c, sem_dc)
        scat_start(val_c, dst_c, sem_sc)

    def pair_body(j, carry):
        idx_wait(2 * j + 1, src_b, ew_b, sem_ib0, sem_ib2)
        pltpu.make_async_copy(y_hbm.at[src_b], rows_b, sem_gb).start()
        half(2 * j, 2 * j + 2,
             src_a, ew_a, dst_a, rows_a, val_a, sem_ga, sem_sa, sem_ia0,
             sem_ia2, sem_da)
        idx_wait(2 * j + 2, src_a, ew_a, sem_ia0, sem_ia2)
        pltpu.make_async_copy(y_hbm.at[src_a], rows_a, sem_ga).start()
        half(2 * j + 1, 2 * j + 3,
             src_b, ew_b, dst_b, rows_b, val_b, sem_gb, sem_sb, sem_ib0,
             sem_ib2, sem_db)
        return carry

    lax.fori_loop(0, 61, pair_body, 0)
    idx_wait(123, src_b, ew_b, sem_ib0, sem_ib2)
    pltpu.make_async_copy(y_hbm.at[src_b], rows_b, sem_gb).start()
    half(122, 124,
         src_a, ew_a, dst_a, rows_a, val_a, sem_ga, sem_sa, sem_ia0,
         sem_ia2, sem_da)
    pltpu.make_async_copy(y_hbm.at[src_b], rows_b, sem_gb).wait()
    scat_wait(val_b, dst_b, sem_sb)
    dst_start(123, dst_b, sem_db)
    scale(ew_b, rows_b, val_b)
    dst_wait(123, dst_b, sem_db)
    scat_start(val_b, dst_b, sem_sb)
    idx_wait(124, src_a, ew_a, sem_ia0, sem_ia2)
    pltpu.make_async_copy(y_hbm.at[src_a], rows_a, sem_ga).start()
    pltpu.make_async_copy(y_hbm.at[src_a], rows_a, sem_ga).wait()
    scat_wait(val_a, dst_a, sem_sa)
    dst_start(124, dst_a, sem_da)
    scale(ew_a, rows_a, val_a)
    dst_wait(124, dst_a, sem_da)
    scat_start(val_a, dst_a, sem_sa)
    scat_wait(val_a, dst_a, sem_sa)
    scat_wait(val_b, dst_b, sem_sb)

    plsc.subcore_barrier()
    pltpu.sync_copy(acc.at[pl.ds(base, ROWS_PER_TILE)],
                    out_hbm.at[c].at[pl.ds(base, ROWS_PER_TILE)])


def _dinv_from(degp_ref):
    deg = degp_ref[0][:, 0:1] + degp_ref[1][:, 0:1] + 1.0
    return lax.rsqrt(deg)


def _tc_a_body(degp_ref, x_ref, w1_ref, y1_ref):
    dinv = _dinv_from(degp_ref)
    xw = jnp.dot(x_ref[...], w1_ref[...], preferred_element_type=jnp.float32)
    y = dinv * xw
    y1_ref[...] = jnp.concatenate([y, jnp.zeros_like(y)], axis=1)


def _tc_b_body(degp_ref, p_ref, y_ref, b_ref, w_ref, out_ref):
    dinv = _dinv_from(degp_ref)
    y = y_ref[...][:, 0:HID]
    agg = p_ref[0][:, 0:HID] + p_ref[1][:, 0:HID]
    h = jnp.maximum(dinv * (agg + y) + b_ref[...], 0.0)
    y2 = dinv * jnp.dot(h, w_ref[...], preferred_element_type=jnp.float32)
    out_ref[...] = jnp.concatenate([y2, jnp.zeros_like(y2)], axis=1)


def _tc_c_body(degp_ref, q_ref, y_ref, b_ref, wfc_ref, bfc_ref, out_ref):
    dinv = _dinv_from(degp_ref)
    y = y_ref[...][:, 0:HID]
    agg = q_ref[0][:, 0:HID] + q_ref[1][:, 0:HID]
    h = jnp.maximum(dinv * (agg + y) + b_ref[...], 0.0)
    out_ref[...] = jnp.dot(h, wfc_ref[...],
                           preferred_element_type=jnp.float32) + bfc_ref[...]


_BN = 2000
_GRID = N_NODES // _BN


def _row_spec(width):
    return pl.BlockSpec((_BN, width), lambda i: (i, 0))


def _pair_spec(width):
    return pl.BlockSpec((NC, _BN, width), lambda i: (0, i, 0))


def _full_spec(shape):
    return pl.BlockSpec(shape, lambda i: tuple(0 for _ in shape))


def kernel(x, edge_index, edge_attr, W1, b1, W2, b2, Wfc, bfc):
    src = edge_index[0]
    dst = edge_index[1]

    degp = _deg_kernel(dst, edge_attr)

    y1 = pl.pallas_call(
        _tc_a_body,
        grid=(_GRID,),
        in_specs=[_pair_spec(128), _row_spec(128), _full_spec(W1.shape)],
        out_specs=_row_spec(2 * HID),
        out_shape=jax.ShapeDtypeStruct((N_NODES, 2 * HID), jnp.float32),
    )(degp, x, W1)

    p = _agg_kernel(y1, src, dst, edge_attr)

    y2 = pl.pallas_call(
        _tc_b_body,
        grid=(_GRID,),
        in_specs=[_pair_spec(128), _pair_spec(128), _row_spec(2 * HID),
                  _full_spec((1, HID)), _full_spec(W2.shape)],
        out_specs=_row_spec(2 * HID),
        out_shape=jax.ShapeDtypeStruct((N_NODES, 2 * HID), jnp.float32),
    )(degp, p, y1, b1.reshape(1, HID), W2)

    q = _agg_kernel(y2, src, dst, edge_attr)

    out = pl.pallas_call(
        _tc_c_body,
        grid=(_GRID,),
        in_specs=[_pair_spec(128), _pair_spec(128), _row_spec(2 * HID),
                  _full_spec((1, HID)), _full_spec(Wfc.shape),
                  _full_spec((1, Wfc.shape[1]))],
        out_specs=_row_spec(Wfc.shape[1]),
        out_shape=jax.ShapeDtypeStruct((N_NODES, Wfc.shape[1]), jnp.float32),
    )(degp, q, y2, b2.reshape(1, HID), Wfc, bfc.reshape(1, -1))

    return out


# speedup vs baseline: 24.6117x; 1.0336x over previous
import functools

import jax
import jax.numpy as jnp
from jax import lax
from jax.experimental import pallas as pl
from jax.experimental.pallas import tpu as pltpu
from jax.experimental.pallas import tpu_sc as plsc

N_NODES = 10000
N_PAD = 10240
N_EDGES = 320000
HID = 64
DEGW = 16
NC = 2
NS = 16
EDGES_PER_TILE = N_EDGES // (NC * NS)
CHUNK = 80
NCHUNKS = EDGES_PER_TILE // CHUNK
ROWS_PER_TILE = N_PAD // NS

_mesh = plsc.VectorSubcoreMesh(core_axis_name="c", subcore_axis_name="s")


@functools.partial(
    pl.kernel,
    out_type=jax.ShapeDtypeStruct((NC, N_PAD, 128), jnp.float32),
    mesh=_mesh,
    scratch_types=[
        pltpu.VMEM((CHUNK,), jnp.int32),
        pltpu.VMEM((CHUNK,), jnp.int32),
        pltpu.VMEM((CHUNK + 16,), jnp.float32),
        pltpu.VMEM((CHUNK + 16,), jnp.float32),
        pltpu.VMEM((CHUNK, 128), jnp.float32),
        pltpu.VMEM((CHUNK, 128), jnp.float32),
        pltpu.VMEM_SHARED((N_PAD, 128), jnp.float32),
    ] + [pltpu.SemaphoreType.DMA] * 6,
)
def _deg_kernel(dst_hbm, ew_hbm, out_hbm,
                dst_a, dst_b, ew_a, ew_b, val_a, val_b, acc,
                sem_sa, sem_sb, sem_ea, sem_eb, sem_da, sem_db):
    c = lax.axis_index("c")
    s = lax.axis_index("s")
    wid = s * NC + c
    base = s * ROWS_PER_TILE
    ebase = wid * EDGES_PER_TILE
    zeros_f = jnp.zeros((16,), jnp.float32)
    lane0 = lax.iota(jnp.int32, 16) == 0

    def zero_plane(v):
        def zero_val(e, carry):
            for j in range(8):
                v[e, pl.ds(j * 16, 16)] = zeros_f
            return carry
        lax.fori_loop(0, CHUNK, zero_val, 0)

    zero_plane(val_a)
    zero_plane(val_b)
    for k in range(ROWS_PER_TILE // CHUNK):
        pltpu.sync_copy(val_a, acc.at[pl.ds(base + k * CHUNK, CHUNK)])
    plsc.subcore_barrier()

    def ew_start(i, ew_v, se):
        off = ebase + i * CHUNK
        pltpu.make_async_copy(ew_hbm.at[pl.ds(off, CHUNK)],
                              ew_v.at[pl.ds(0, CHUNK)], se).start()

    def ew_wait(i, ew_v, se):
        off = ebase + i * CHUNK
        pltpu.make_async_copy(ew_hbm.at[pl.ds(off, CHUNK)],
                              ew_v.at[pl.ds(0, CHUNK)], se).wait()

    def dst_start(i, dst_v, sd):
        off = ebase + i * CHUNK
        pltpu.make_async_copy(dst_hbm.at[pl.ds(off, CHUNK)], dst_v, sd).start()

    def dst_wait(i, dst_v, sd):
        off = ebase + i * CHUNK
        pltpu.make_async_copy(dst_hbm.at[pl.ds(off, CHUNK)], dst_v, sd).wait()

    def fill(ew_v, val_v):
        def fill_body(e4, carry2):
            e = 4 * e4
            ws = [ew_v[pl.ds(e + u, 16)][0] for u in range(4)]
            for u in range(4):
                val_v[e + u, pl.ds(0, 16)] = jnp.where(
                    lane0, zeros_f + ws[u], zeros_f)
            return carry2
        lax.fori_loop(0, CHUNK // 4, fill_body, 0)

    def half(i_cur, i_nxt, ew_c, dst_c, val_c, sem_ec, sem_sc, sem_dc):
        ew_wait(i_cur, ew_c, sem_ec)
        pltpu.make_async_copy(val_c, acc.at[dst_c], sem_sc).wait()
        dst_start(i_cur, dst_c, sem_dc)
        fill(ew_c, val_c)
        if i_nxt is not None:
            ew_start(i_nxt, ew_c, sem_ec)
        dst_wait(i_cur, dst_c, sem_dc)
        pltpu.async_copy(val_c, acc.at[dst_c], sem_sc, add=True)

    ew_start(0, ew_a, sem_ea)
    ew_start(1, ew_b, sem_eb)
    pltpu.sync_copy(dst_hbm.at[pl.ds(ebase, CHUNK)], dst_a)
    pltpu.sync_copy(dst_hbm.at[pl.ds(ebase + CHUNK, CHUNK)], dst_b)
    pltpu.async_copy(val_a, acc.at[dst_a], sem_sa, add=True)
    pltpu.async_copy(val_b, acc.at[dst_b], sem_sb, add=True)

    def pair_body(j, carry):
        half(2 * j, 2 * j + 2, ew_a, dst_a, val_a, sem_ea, sem_sa, sem_da)
        half(2 * j + 1, 2 * j + 3, ew_b, dst_b, val_b, sem_eb, sem_sb, sem_db)
        return carry

    lax.fori_loop(0, 61, pair_body, 0)
    half(122, 124, ew_a, dst_a, val_a, sem_ea, sem_sa, sem_da)
    half(123, None, ew_b, dst_b, val_b, sem_eb, sem_sb, sem_db)
    half(124, None, ew_a, dst_a, val_a, sem_ea, sem_sa, sem_da)
    pltpu.make_async_copy(val_a, acc.at[dst_a], sem_sa).wait()
    pltpu.make_async_copy(val_b, acc.at[dst_b], sem_sb).wait()

    plsc.subcore_barrier()
    pltpu.sync_copy(acc.at[pl.ds(base, ROWS_PER_TILE)],
                    out_hbm.at[c].at[pl.ds(base, ROWS_PER_TILE)])


@functools.partial(
    pl.kernel,
    out_type=jax.ShapeDtypeStruct((NC, N_PAD, 128), jnp.float32),
    mesh=_mesh,
    scratch_types=[
        pltpu.VMEM((CHUNK,), jnp.int32),
        pltpu.VMEM((CHUNK,), jnp.int32),
        pltpu.VMEM((CHUNK,), jnp.int32),
        pltpu.VMEM((CHUNK,), jnp.int32),
        pltpu.VMEM((CHUNK + 16,), jnp.float32),
        pltpu.VMEM((CHUNK + 16,), jnp.float32),
        pltpu.VMEM((CHUNK, 128), jnp.float32),
        pltpu.VMEM((CHUNK, 128), jnp.float32),
        pltpu.VMEM((CHUNK, 128), jnp.float32),
        pltpu.VMEM((CHUNK, 128), jnp.float32),
        pltpu.VMEM_SHARED((N_PAD, 128), jnp.float32),
    ] + [pltpu.SemaphoreType.DMA] * 10,
)
def _agg_kernel(y_hbm, src_hbm, dst_hbm, ew_hbm, out_hbm,
                src_a, src_b, dst_a, dst_b, ew_a, ew_b,
                rows_a, rows_b, val_a, val_b, acc,
                sem_ga, sem_gb, sem_sa, sem_sb,
                sem_ia0, sem_ia2, sem_ib0, sem_ib2, sem_da, sem_db):
    c = lax.axis_index("c")
    s = lax.axis_index("s")
    wid = s * NC + c
    base = s * ROWS_PER_TILE
    ebase = wid * EDGES_PER_TILE
    zeros_f = jnp.zeros((16,), jnp.float32)

    def zero_plane(v):
        def zero_val(e, carry):
            for j in range(8):
                v[e, pl.ds(j * 16, 16)] = zeros_f
            return carry
        lax.fori_loop(0, CHUNK, zero_val, 0)

    zero_plane(val_a)
    zero_plane(val_b)
    for k in range(ROWS_PER_TILE // CHUNK):
        pltpu.sync_copy(val_a, acc.at[pl.ds(base + k * CHUNK, CHUNK)])
    plsc.subcore_barrier()

    def idx_start(i, src_v, ew_v, s0, s2):
        off = ebase + i * CHUNK
        pltpu.make_async_copy(src_hbm.at[pl.ds(off, CHUNK)], src_v, s0).start()
        pltpu.make_async_copy(ew_hbm.at[pl.ds(off, CHUNK)],
                              ew_v.at[pl.ds(0, CHUNK)], s2).start()

    def idx_wait(i, src_v, ew_v, s0, s2):
        off = ebase + i * CHUNK
        pltpu.make_async_copy(src_hbm.at[pl.ds(off, CHUNK)], src_v, s0).wait()
        pltpu.make_async_copy(ew_hbm.at[pl.ds(off, CHUNK)],
                              ew_v.at[pl.ds(0, CHUNK)], s2).wait()

    def dst_start(i, dst_v, sd):
        off = ebase + i * CHUNK
        pltpu.make_async_copy(dst_hbm.at[pl.ds(off, CHUNK)], dst_v, sd).start()

    def dst_wait(i, dst_v, sd):
        off = ebase + i * CHUNK
        pltpu.make_async_copy(dst_hbm.at[pl.ds(off, CHUNK)], dst_v, sd).wait()

    def scat_start(val_v, dst_v, sem_s):
        pltpu.async_copy(val_v, acc.at[dst_v], sem_s, add=True)

    def scat_wait(val_v, dst_v, sem_s):
        pltpu.make_async_copy(val_v, acc.at[dst_v], sem_s).wait()

    def scale(ew_v, rows_v, val_v):
        def scale_body(e4, carry2):
            e = 4 * e4
            ws = [zeros_f + ew_v[pl.ds(e + u, 16)][0] for u in range(4)]
            for u in range(4):
                for j in range(HID // 16):
                    val_v[e + u, pl.ds(j * 16, 16)] = (
                        rows_v[e + u, pl.ds(j * 16, 16)] * ws[u])
            return carry2
        lax.fori_loop(0, CHUNK // 4, scale_body, 0)

    idx_start(0, src_a, ew_a, sem_ia0, sem_ia2)
    idx_wait(0, src_a, ew_a, sem_ia0, sem_ia2)
    pltpu.make_async_copy(y_hbm.at[src_a], rows_a, sem_ga).start()
    pltpu.sync_copy(dst_hbm.at[pl.ds(ebase, CHUNK)], dst_a)
    pltpu.sync_copy(dst_hbm.at[pl.ds(ebase + CHUNK, CHUNK)], dst_b)
    scat_start(val_a, dst_a, sem_sa)
    scat_start(val_b, dst_b, sem_sb)
    idx_start(1, src_b, ew_b, sem_ib0, sem_ib2)

    def src_start(i, src_v, s0):
        off = ebase + i * CHUNK
        pltpu.make_async_copy(src_hbm.at[pl.ds(off, CHUNK)], src_v, s0).start()

    def ew_start(i, ew_v, s2):
        off = ebase + i * CHUNK
        pltpu.make_async_copy(ew_hbm.at[pl.ds(off, CHUNK)],
                              ew_v.at[pl.ds(0, CHUNK)], s2).start()

    def half(i_cur, i_nxt,
             src_c, ew_c, dst_c, rows_c, val_c, sem_gc, sem_sc, sem_ic0,
             sem_ic2, sem_dc):
        pltpu.make_async_copy(y_hbm.at[src_c], rows_c, sem_gc).wait()
        src_start(i_nxt, src_c, sem_ic0)
        scat_wait(val_c, dst_c, sem_sc)
        dst_start(i_cur, dst_c, sem_dc)
        scale(ew_c, rows_c, val_c)
        ew_start(i_nxt, ew_c, sem_ic2)
        dst_wait(i_cur, dst_c, sem_dc)
        scat_start(val_c, dst_c, sem_sc)

    def pair_body(j, carry):
        idx_wait(2 * j + 1, src_b, ew_b, sem_ib0, sem_ib2)
        pltpu.make_async_copy(y_hbm.at[src_b], rows_b, sem_gb).start()
        half(2 * j, 2 * j + 2,
             src_a, ew_a, dst_a, rows_a, val_a, sem_ga, sem_sa, sem_ia0,
             sem_ia2, sem_da)
        idx_wait(2 * j + 2, src_a, ew_a, sem_ia0, sem_ia2)
        pltpu.make_async_copy(y_hbm.at[src_a], rows_a, sem_ga).start()
        half(2 * j + 1, 2 * j + 3,
             src_b, ew_b, dst_b, rows_b, val_b, sem_gb, sem_sb, sem_ib0,
             sem_ib2, sem_db)
        return carry

    lax.fori_loop(0, 61, pair_body, 0)
    idx_wait(123, src_b, ew_b, sem_ib0, sem_ib2)
    pltpu.make_async_copy(y_hbm.at[src_b], rows_b, sem_gb).start()
    half(122, 124,
         src_a, ew_a, dst_a, rows_a, val_a, sem_ga, sem_sa, sem_ia0,
         sem_ia2, sem_da)
    pltpu.make_async_copy(y_hbm.at[src_b], rows_b, sem_gb).wait()
    scat_wait(val_b, dst_b, sem_sb)
    dst_start(123, dst_b, sem_db)
    scale(ew_b, rows_b, val_b)
    dst_wait(123, dst_b, sem_db)
    scat_start(val_b, dst_b, sem_sb)
    idx_wait(124, src_a, ew_a, sem_ia0, sem_ia2)
    pltpu.make_async_copy(y_hbm.at[src_a], rows_a, sem_ga).start()
    pltpu.make_async_copy(y_hbm.at[src_a], rows_a, sem_ga).wait()
    scat_wait(val_a, dst_a, sem_sa)
    dst_start(124, dst_a, sem_da)
    scale(ew_a, rows_a, val_a)
    dst_wait(124, dst_a, sem_da)
    scat_start(val_a, dst_a, sem_sa)
    scat_wait(val_a, dst_a, sem_sa)
    scat_wait(val_b, dst_b, sem_sb)

    plsc.subcore_barrier()
    pltpu.sync_copy(acc.at[pl.ds(base, ROWS_PER_TILE)],
                    out_hbm.at[c].at[pl.ds(base, ROWS_PER_TILE)])


def _dinv_from(degp_ref):
    deg = degp_ref[0][:, 0:1] + degp_ref[1][:, 0:1] + 1.0
    return lax.rsqrt(deg)


def _tc_a_body(degp_ref, x_ref, w1_ref, y1_ref):
    dinv = _dinv_from(degp_ref)
    xw = jnp.dot(x_ref[...], w1_ref[...], preferred_element_type=jnp.float32)
    y = dinv * xw
    y1_ref[...] = jnp.concatenate([y, jnp.zeros_like(y)], axis=1)


def _tc_b_body(degp_ref, p_ref, y_ref, b_ref, w_ref, out_ref):
    dinv = _dinv_from(degp_ref)
    y = y_ref[...][:, 0:HID]
    agg = p_ref[0][:, 0:HID] + p_ref[1][:, 0:HID]
    h = jnp.maximum(dinv * (agg + y) + b_ref[...], 0.0)
    y2 = dinv * jnp.dot(h, w_ref[...], preferred_element_type=jnp.float32)
    out_ref[...] = jnp.concatenate([y2, jnp.zeros_like(y2)], axis=1)


def _tc_c_body(degp_ref, q_ref, y_ref, b_ref, wfc_ref, bfc_ref, out_ref):
    dinv = _dinv_from(degp_ref)
    y = y_ref[...][:, 0:HID]
    agg = q_ref[0][:, 0:HID] + q_ref[1][:, 0:HID]
    h = jnp.maximum(dinv * (agg + y) + b_ref[...], 0.0)
    out_ref[...] = jnp.dot(h, wfc_ref[...],
                           preferred_element_type=jnp.float32) + bfc_ref[...]


_BN = 2000
_GRID = N_NODES // _BN


def _row_spec(width):
    return pl.BlockSpec((_BN, width), lambda i: (i, 0))


def _pair_spec(width):
    return pl.BlockSpec((NC, _BN, width), lambda i: (0, i, 0))


def _full_spec(shape):
    return pl.BlockSpec(shape, lambda i: tuple(0 for _ in shape))


def kernel(x, edge_index, edge_attr, W1, b1, W2, b2, Wfc, bfc):
    src = edge_index[0]
    dst = edge_index[1]

    degp = _deg_kernel(dst, edge_attr)

    y1 = pl.pallas_call(
        _tc_a_body,
        grid=(_GRID,),
        in_specs=[_pair_spec(128), _row_spec(128), _full_spec(W1.shape)],
        out_specs=_row_spec(2 * HID),
        out_shape=jax.ShapeDtypeStruct((N_NODES, 2 * HID), jnp.float32),
    )(degp, x, W1)

    p = _agg_kernel(y1, src, dst, edge_attr)

    y2 = pl.pallas_call(
        _tc_b_body,
        grid=(_GRID,),
        in_specs=[_pair_spec(128), _pair_spec(128), _row_spec(2 * HID),
                  _full_spec((1, HID)), _full_spec(W2.shape)],
        out_specs=_row_spec(2 * HID),
        out_shape=jax.ShapeDtypeStruct((N_NODES, 2 * HID), jnp.float32),
    )(degp, p, y1, b1.reshape(1, HID), W2)

    q = _agg_kernel(y2, src, dst, edge_attr)

    out = pl.pallas_call(
        _tc_c_body,
        grid=(_GRID,),
        in_specs=[_pair_spec(128), _pair_spec(128), _row_spec(2 * HID),
                  _full_spec((1, HID)), _full_spec(Wfc.shape),
                  _full_spec((1, Wfc.shape[1]))],
        out_specs=_row_spec(Wfc.shape[1]),
        out_shape=jax.ShapeDtypeStruct((N_NODES, Wfc.shape[1]), jnp.float32),
    )(degp, q, y2, b2.reshape(1, HID), Wfc, bfc.reshape(1, -1))

    return out
